# baseline (speedup 1.0000x reference)
import functools

import jax
import jax.numpy as jnp
from jax import lax
from jax.experimental import pallas as pl
from jax.experimental.pallas import tpu as pltpu
from jax.experimental.pallas import tpu_sc as plsc

N1 = 5001
D = 64
L = 50
B = 256

ROW_BLK = 256
SEQ_BLK = 8


def _proj_body(emb_ref, embT_ref, Wi_ref, WiT_ref, a1_ref, a2T_ref,
               h_ref, wh1_ref, wh2r_ref):
    emb = emb_ref[...]
    h = jnp.dot(emb, Wi_ref[...], preferred_element_type=jnp.float32)
    h_ref[...] = h
    wh1_ref[...] = jnp.dot(h, a1_ref[...], preferred_element_type=jnp.float32)
    v = jnp.dot(a2T_ref[...], WiT_ref[...],
                preferred_element_type=jnp.float32)
    wh2r_ref[...] = jnp.dot(v, embT_ref[...],
                            preferred_element_type=jnp.float32)


def _run_proj(item_emb, W_item, a_item):
    emb_T = jnp.transpose(item_emb)
    Wi_T = jnp.transpose(W_item)
    a1 = a_item[:D]
    a2T = jnp.transpose(a_item[D:])
    return pl.pallas_call(
        _proj_body,
        out_shape=(
            jax.ShapeDtypeStruct((N1, D), jnp.float32),
            jax.ShapeDtypeStruct((N1, 1), jnp.float32),
            jax.ShapeDtypeStruct((1, N1), jnp.float32),
        ),
    )(item_emb, emb_T, W_item, Wi_T, a1, a2T)


def _graph_body(adj_ref, wh1_ref, wh2r_ref, h_ref, cc_ref, cn_ref, emb_ref,
                t1_ref, t2_ref):
    a = adj_ref[...]
    e = wh1_ref[...] + wh2r_ref[...]
    e = jnp.where(e >= 0.0, e, 0.01 * e)
    ex = jnp.where(a > 0.0, jnp.exp(e), 0.0)
    s = jnp.sum(ex, axis=1, keepdims=True)
    srecip = 1.0 / jnp.where(s > 0.0, s, float(N1))
    attn = jnp.where(s > 0.0, ex, 1.0) * srecip
    h = h_ref[...].astype(jnp.bfloat16)
    gat = jnp.dot(attn.astype(jnp.bfloat16), h,
                  preferred_element_type=jnp.float32)
    rs = jnp.sum(a, axis=1, keepdims=True)
    ti = jnp.dot(a.astype(jnp.bfloat16), h,
                 preferred_element_type=jnp.float32) / (rs + 1e-8)
    g1 = (jnp.dot(gat, cc_ref[...], preferred_element_type=jnp.float32)
          + jnp.dot(ti, cn_ref[...], preferred_element_type=jnp.float32))
    t1_ref[...] = jnp.concatenate([gat, ti], axis=1)
    t2_ref[...] = jnp.concatenate([g1, emb_ref[...]], axis=1)


def _run_graph(adj, wh1, wh2r, h, co_center, co_neighbor, item_emb):
    grid = (pl.cdiv(N1, ROW_BLK),)
    return pl.pallas_call(
        _graph_body,
        grid=grid,
        in_specs=[
            pl.BlockSpec((ROW_BLK, N1), lambda i: (i, 0)),
            pl.BlockSpec((ROW_BLK, 1), lambda i: (i, 0)),
            pl.BlockSpec((1, N1), lambda i: (0, 0)),
            pl.BlockSpec((N1, D), lambda i: (0, 0)),
            pl.BlockSpec((D, D), lambda i: (0, 0)),
            pl.BlockSpec((D, D), lambda i: (0, 0)),
            pl.BlockSpec((ROW_BLK, D), lambda i: (i, 0)),
        ],
        out_specs=(
            pl.BlockSpec((ROW_BLK, 2 * D), lambda i: (i, 0)),
            pl.BlockSpec((ROW_BLK, 2 * D), lambda i: (i, 0)),
        ),
        out_shape=(
            jax.ShapeDtypeStruct((N1, 2 * D), jnp.float32),
            jax.ShapeDtypeStruct((N1, 2 * D), jnp.float32),
        ),
    )(adj, wh1, wh2r, h, co_center, co_neighbor, item_emb)


_NW = 32
_TOK = B * L
_PER_W = _TOK // _NW
_CHUNK = 80
_NCH = _PER_W // _CHUNK


def _gather_body(t0, t1, idx_hbm, o0, o1, idx_v, rows_v, sem):
    nc = 2
    wid = lax.axis_index("s") * nc + lax.axis_index("c")
    pltpu.sync_copy(idx_hbm.at[wid], idx_v)
    for tab, out in ((t0, o0), (t1, o1)):
        handles = [
            pltpu.async_copy(tab.at[idx_v.at[j]],
                             rows_v.at[pl.ds(j * _CHUNK, _CHUNK)], sem)
            for j in range(_NCH)
        ]
        for hd in handles:
            hd.wait()
        pltpu.sync_copy(rows_v, out.at[pl.ds(wid * _PER_W, _PER_W)])


def _run_gather(t1, t2, idx_flat):
    idx3 = idx_flat.reshape(_NW, _NCH, _CHUNK)
    mesh = plsc.VectorSubcoreMesh(core_axis_name="c", subcore_axis_name="s")
    out_t = tuple(jax.ShapeDtypeStruct((_TOK, 2 * D), jnp.float32)
                  for _ in range(2))
    fn = functools.partial(
        pl.kernel,
        mesh=mesh,
        out_type=out_t,
        scratch_types=[
            pltpu.VMEM((_NCH, _CHUNK), jnp.int32),
            pltpu.VMEM((_PER_W, 2 * D), jnp.float32),
            pltpu.SemaphoreType.DMA,
        ],
    )(_gather_body)
    return fn(t1, t2, idx3)


TOK_BLK = 512


def _seq1_body(p1_ref, p2_ref, ls_ref, pos_ref, W1_ref, W2_ref,
               seqs_ref, e1_ref, e2_ref):
    p1 = p1_ref[...]
    p2 = p2_ref[...]
    gat, tr = p1[:, :D], p1[:, D:]
    g1, se = p2[:, :D], p2[:, D:]
    coff = jax.nn.sigmoid(g1)
    seqs = coff * gat + (1.0 - coff) * tr + se
    keep = (ls_ref[...] != 0).astype(jnp.float32)
    sp = seqs + pos_ref[...] * keep
    m1 = jnp.dot(sp, W1_ref[...], preferred_element_type=jnp.float32)
    m2 = jnp.dot(sp, W2_ref[...], preferred_element_type=jnp.float32)
    seqs_ref[...] = seqs
    e1_ref[...] = jnp.exp(-m1)
    e2_ref[...] = jnp.exp(-m2)


def _run_seq1(p1g, p2g, log_seqs, pos_emb, W_1, W_2):
    ls2 = log_seqs.reshape(_TOK, 1)
    pos_t = jnp.tile(pos_emb, (B, 1))
    grid = (_TOK // TOK_BLK,)
    blk = lambda w: pl.BlockSpec((TOK_BLK, w), lambda i: (i, 0))
    full = lambda shape: pl.BlockSpec(shape, lambda i: (0, 0))
    out = jax.ShapeDtypeStruct((_TOK, D), jnp.float32)
    return pl.pallas_call(
        _seq1_body,
        grid=grid,
        in_specs=[blk(2 * D), blk(2 * D), blk(1), blk(D),
                  full((D, D)), full((D, D))],
        out_specs=(blk(D), blk(D), blk(D)),
        out_shape=(out, out, out),
    )(p1g, p2g, ls2, pos_t, W_1, W_2)


def _seq2_body(seqs_ref, e1_ref, e2f_ref, bcol_ref,
               c1w_ref, c1b_ref, c2w_ref, c2b_ref,
               upw_ref, upb_ref, gw_ref, gb_ref, dw_ref, db_ref, out_ref):
    ri = lax.broadcasted_iota(jnp.int32, (L, L), 0)
    ci = lax.broadcasted_iota(jnp.int32, (L, L), 1)
    causal = ci <= ri
    srow = lax.broadcasted_iota(jnp.int32, (L * D, L), 0)
    jcol = lax.broadcasted_iota(jnp.int32, (L * D, L), 1)
    btile = jnp.tile(bcol_ref[...], (L, 1))
    B3 = (jnp.where(srow // D == jcol, 1.0, 0.0) * btile).astype(jnp.bfloat16)
    seqs = seqs_ref[...]
    e1a = e1_ref[...]
    e2f = e2f_ref[...]
    for bi in range(SEQ_BLK):
        e1t = jnp.tile(e1a[bi], (1, L))
        e2t = jnp.broadcast_to(e2f[bi][None, :], (L, L * D))
        a2 = (1.0 / (1.0 + e1t * e2t)).astype(jnp.bfloat16)
        s = jnp.dot(a2, B3, preferred_element_type=jnp.float32)
        s = jnp.where(causal, s, 0.0)
        fin = jnp.dot(s, seqs[bi], preferred_element_type=jnp.float32)
        hh = jnp.maximum(
            jnp.dot(fin, c1w_ref[...], preferred_element_type=jnp.float32)
            + c1b_ref[...], 0.0)
        hh = jnp.dot(hh, c2w_ref[...], preferred_element_type=jnp.float32) \
            + c2b_ref[...]
        fin = fin + hh
        y_up = jnp.dot(fin, upw_ref[...], preferred_element_type=jnp.float32) \
            + upb_ref[...]
        gate = jnp.maximum(
            jnp.dot(fin, gw_ref[...], preferred_element_type=jnp.float32)
            + gb_ref[...], 0.0)
        dn = jnp.maximum(
            jnp.dot(gate * y_up, dw_ref[...], preferred_element_type=jnp.float32)
            + db_ref[...], 0.0)
        out_ref[bi, :, :] = fin + dn


def _run_seq2(seqs, e1, e2, b, conv1_w, conv1_b, conv2_w, conv2_b,
              up_w, up_b, gate_w, gate_b, down_w, down_b):
    seqs3 = seqs.reshape(B, L, D)
    e13 = e1.reshape(B, L, D)
    e2f = e2.reshape(B, L * D)
    r1 = lambda x: x.reshape(1, -1)
    grid = (B // SEQ_BLK,)
    tok = pl.BlockSpec((SEQ_BLK, L, D), lambda i: (i, 0, 0))
    full = lambda shape: pl.BlockSpec(shape, lambda i: tuple(0 for _ in shape))
    return pl.pallas_call(
        _seq2_body,
        grid=grid,
        in_specs=[
            tok, tok,
            pl.BlockSpec((SEQ_BLK, L * D), lambda i: (i, 0)),
            full((D, 1)),
            full((D, D)), full((1, D)), full((D, D)), full((1, D)),
            full((D, 2 * D)), full((1, 2 * D)),
            full((D, 2 * D)), full((1, 2 * D)),
            full((2 * D, D)), full((1, D)),
        ],
        out_specs=pl.BlockSpec((SEQ_BLK, L, D), lambda i: (i, 0, 0)),
        out_shape=jax.ShapeDtypeStruct((B, L, D), jnp.float32),
    )(seqs3, e13, e2f, b,
      conv1_w, r1(conv1_b), conv2_w, r1(conv2_b),
      up_w, r1(up_b), gate_w, r1(gate_b), down_w, r1(down_b))


def kernel(log_seqs, item_emb, pos_emb, W_item, a_item, W_1, W_2, b,
           co_center, co_neighbor, conv1_w, conv1_b, conv2_w, conv2_b,
           up_w, up_b, gate_w, gate_b, down_w, down_b, adj):
    log_seqs = log_seqs.astype(jnp.int32)
    h, wh1, wh2r = _run_proj(item_emb, W_item, a_item)
    tab1, tab2 = _run_graph(adj, wh1, wh2r, h, co_center, co_neighbor,
                            item_emb)
    idx_flat = log_seqs.reshape(-1)
    g1g, g2g = _run_gather(tab1, tab2, idx_flat)
    seqs, e1, e2 = _run_seq1(g1g, g2g, log_seqs, pos_emb, W_1, W_2)
    return _run_seq2(seqs, e1, e2, b, conv1_w, conv1_b, conv2_w, conv2_b,
                     up_w, up_b, gate_w, gate_b, down_w, down_b)


# speedup vs baseline: 1.3429x; 1.3429x over previous
import functools

import jax
import jax.numpy as jnp
from jax import lax
from jax.experimental import pallas as pl
from jax.experimental.pallas import tpu as pltpu
from jax.experimental.pallas import tpu_sc as plsc

N1 = 5001
D = 64
L = 50
B = 256

ROW_BLK = 256
SEQ_BLK = 8


def _proj_body(emb_ref, embT_ref, Wi_ref, WiT_ref, a1_ref, a2T_ref,
               h_ref, wh1_ref, wh2r_ref):
    emb = emb_ref[...]
    h = jnp.dot(emb, Wi_ref[...], preferred_element_type=jnp.float32)
    h_ref[...] = h
    wh1_ref[...] = jnp.dot(h, a1_ref[...], preferred_element_type=jnp.float32)
    v = jnp.dot(a2T_ref[...], WiT_ref[...],
                preferred_element_type=jnp.float32)
    wh2r_ref[...] = jnp.dot(v, embT_ref[...],
                            preferred_element_type=jnp.float32)


def _run_proj(item_emb, W_item, a_item):
    emb_T = jnp.transpose(item_emb)
    Wi_T = jnp.transpose(W_item)
    a1 = a_item[:D]
    a2T = jnp.transpose(a_item[D:])
    return pl.pallas_call(
        _proj_body,
        out_shape=(
            jax.ShapeDtypeStruct((N1, D), jnp.float32),
            jax.ShapeDtypeStruct((N1, 1), jnp.float32),
            jax.ShapeDtypeStruct((1, N1), jnp.float32),
        ),
    )(item_emb, emb_T, W_item, Wi_T, a1, a2T)


def _graph_body(adj_ref, wh1_ref, wh2r_ref, h_ref, cc_ref, cn_ref, emb_ref,
                t1_ref, t2_ref):
    a = adj_ref[...]
    e = wh1_ref[...] + wh2r_ref[...]
    e = jnp.where(e >= 0.0, e, 0.01 * e)
    ex = jnp.where(a > 0.0, jnp.exp(e), 0.0)
    s = jnp.sum(ex, axis=1, keepdims=True)
    srecip = 1.0 / jnp.where(s > 0.0, s, float(N1))
    attn = jnp.where(s > 0.0, ex, 1.0) * srecip
    h = h_ref[...]
    gat = jnp.dot(attn, h, preferred_element_type=jnp.float32)
    rs = jnp.sum(a, axis=1, keepdims=True)
    ti = jnp.dot(a, h, preferred_element_type=jnp.float32) / (rs + 1e-8)
    g1 = (jnp.dot(gat, cc_ref[...], preferred_element_type=jnp.float32)
          + jnp.dot(ti, cn_ref[...], preferred_element_type=jnp.float32))
    t1_ref[...] = jnp.concatenate([gat, ti], axis=1)
    t2_ref[...] = jnp.concatenate([g1, emb_ref[...]], axis=1)


def _run_graph(adj, wh1, wh2r, h, co_center, co_neighbor, item_emb):
    grid = (pl.cdiv(N1, ROW_BLK),)
    return pl.pallas_call(
        _graph_body,
        grid=grid,
        in_specs=[
            pl.BlockSpec((ROW_BLK, N1), lambda i: (i, 0)),
            pl.BlockSpec((ROW_BLK, 1), lambda i: (i, 0)),
            pl.BlockSpec((1, N1), lambda i: (0, 0)),
            pl.BlockSpec((N1, D), lambda i: (0, 0)),
            pl.BlockSpec((D, D), lambda i: (0, 0)),
            pl.BlockSpec((D, D), lambda i: (0, 0)),
            pl.BlockSpec((ROW_BLK, D), lambda i: (i, 0)),
        ],
        out_specs=(
            pl.BlockSpec((ROW_BLK, 2 * D), lambda i: (i, 0)),
            pl.BlockSpec((ROW_BLK, 2 * D), lambda i: (i, 0)),
        ),
        out_shape=(
            jax.ShapeDtypeStruct((N1, 2 * D), jnp.float32),
            jax.ShapeDtypeStruct((N1, 2 * D), jnp.float32),
        ),
    )(adj, wh1, wh2r, h, co_center, co_neighbor, item_emb)


_NW = 32
_TOK = B * L
_PER_W = _TOK // _NW
_CHUNK = 80
_NCH = _PER_W // _CHUNK


def _gather_body(t0, t1, idx_hbm, o0, o1, idx_v, rows_v, sem):
    nc = 2
    wid = lax.axis_index("s") * nc + lax.axis_index("c")
    pltpu.sync_copy(idx_hbm.at[wid], idx_v)
    for tab, out in ((t0, o0), (t1, o1)):
        handles = [
            pltpu.async_copy(tab.at[idx_v.at[j]],
                             rows_v.at[pl.ds(j * _CHUNK, _CHUNK)], sem)
            for j in range(_NCH)
        ]
        for hd in handles:
            hd.wait()
        pltpu.sync_copy(rows_v, out.at[pl.ds(wid * _PER_W, _PER_W)])


def _run_gather(t1, t2, idx_flat):
    idx3 = idx_flat.reshape(_NW, _NCH, _CHUNK)
    mesh = plsc.VectorSubcoreMesh(core_axis_name="c", subcore_axis_name="s")
    out_t = tuple(jax.ShapeDtypeStruct((_TOK, 2 * D), jnp.float32)
                  for _ in range(2))
    fn = functools.partial(
        pl.kernel,
        mesh=mesh,
        out_type=out_t,
        scratch_types=[
            pltpu.VMEM((_NCH, _CHUNK), jnp.int32),
            pltpu.VMEM((_PER_W, 2 * D), jnp.float32),
            pltpu.SemaphoreType.DMA,
        ],
    )(_gather_body)
    return fn(t1, t2, idx3)


TOK_BLK = 512


def _seq1_body(p1_ref, p2_ref, ls_ref, pos_ref, W1_ref, W2_ref,
               seqs_ref, e1_ref, e2_ref):
    p1 = p1_ref[...]
    p2 = p2_ref[...]
    gat, tr = p1[:, :D], p1[:, D:]
    g1, se = p2[:, :D], p2[:, D:]
    coff = jax.nn.sigmoid(g1)
    seqs = coff * gat + (1.0 - coff) * tr + se
    keep = (ls_ref[...] != 0).astype(jnp.float32)
    sp = seqs + pos_ref[...] * keep
    m1 = jnp.dot(sp, W1_ref[...], preferred_element_type=jnp.float32)
    m2 = jnp.dot(sp, W2_ref[...], preferred_element_type=jnp.float32)
    seqs_ref[...] = seqs
    e1_ref[...] = jnp.exp(-m1)
    e2_ref[...] = jnp.exp(-m2)


def _run_seq1(p1g, p2g, log_seqs, pos_emb, W_1, W_2):
    ls2 = log_seqs.reshape(_TOK, 1)
    pos_t = jnp.tile(pos_emb, (B, 1))
    grid = (_TOK // TOK_BLK,)
    blk = lambda w: pl.BlockSpec((TOK_BLK, w), lambda i: (i, 0))
    full = lambda shape: pl.BlockSpec(shape, lambda i: (0, 0))
    out = jax.ShapeDtypeStruct((_TOK, D), jnp.float32)
    return pl.pallas_call(
        _seq1_body,
        grid=grid,
        in_specs=[blk(2 * D), blk(2 * D), blk(1), blk(D),
                  full((D, D)), full((D, D))],
        out_specs=(blk(D), blk(D), blk(D)),
        out_shape=(out, out, out),
    )(p1g, p2g, ls2, pos_t, W_1, W_2)


def _seq2_body(seqs_ref, e1_ref, e2f_ref, bcol_ref,
               c1w_ref, c1b_ref, c2w_ref, c2b_ref,
               upw_ref, upb_ref, gw_ref, gb_ref, dw_ref, db_ref, out_ref):
    ri = lax.broadcasted_iota(jnp.int32, (SEQ_BLK * L, L), 0)
    ci = lax.broadcasted_iota(jnp.int32, (SEQ_BLK * L, L), 1)
    causal_t = ci <= ri % L
    srow = lax.broadcasted_iota(jnp.int32, (L * D, L), 0)
    jcol = lax.broadcasted_iota(jnp.int32, (L * D, L), 1)
    btile = jnp.tile(bcol_ref[...], (L, 1))
    B3 = jnp.where(srow // D == jcol, 1.0, 0.0) * btile
    tk = lax.broadcasted_iota(jnp.int32, (D, L * D), 0)
    tc = lax.broadcasted_iota(jnp.int32, (D, L * D), 1)
    TILE = jnp.where(tc % D == tk, 1.0, 0.0)
    a2s = []
    for bi in range(SEQ_BLK):
        e1t = jnp.dot(e1_ref[bi], TILE, preferred_element_type=jnp.float32)
        e2t = jnp.broadcast_to(e2f_ref[pl.ds(bi, 1), :], (L, L * D))
        a2s.append(1.0 / (1.0 + e1t * e2t))
    a2a = jnp.concatenate(a2s, axis=0)
    sa = jnp.dot(a2a, B3, preferred_element_type=jnp.float32)
    sa = jnp.where(causal_t, sa, 0.0)
    fins = [
        jnp.dot(sa[bi * L:(bi + 1) * L], seqs_ref[bi],
                preferred_element_type=jnp.float32)
        for bi in range(SEQ_BLK)
    ]
    fin = jnp.concatenate(fins, axis=0)
    hh = jnp.maximum(
        jnp.dot(fin, c1w_ref[...], preferred_element_type=jnp.float32)
        + c1b_ref[...], 0.0)
    hh = jnp.dot(hh, c2w_ref[...], preferred_element_type=jnp.float32) \
        + c2b_ref[...]
    fin = fin + hh
    y_up = jnp.dot(fin, upw_ref[...], preferred_element_type=jnp.float32) \
        + upb_ref[...]
    gate = jnp.maximum(
        jnp.dot(fin, gw_ref[...], preferred_element_type=jnp.float32)
        + gb_ref[...], 0.0)
    dn = jnp.maximum(
        jnp.dot(gate * y_up, dw_ref[...], preferred_element_type=jnp.float32)
        + db_ref[...], 0.0)
    out_ref[...] = (fin + dn).reshape(SEQ_BLK, L, D)


def _run_seq2(seqs, e1, e2, b, conv1_w, conv1_b, conv2_w, conv2_b,
              up_w, up_b, gate_w, gate_b, down_w, down_b):
    seqs3 = seqs.reshape(B, L, D)
    e13 = e1.reshape(B, L, D)
    e2f = e2.reshape(B, L * D)
    r1 = lambda x: x.reshape(1, -1)
    grid = (B // SEQ_BLK,)
    tok = pl.BlockSpec((SEQ_BLK, L, D), lambda i: (i, 0, 0))
    full = lambda shape: pl.BlockSpec(shape, lambda i: tuple(0 for _ in shape))
    return pl.pallas_call(
        _seq2_body,
        grid=grid,
        in_specs=[
            tok, tok,
            pl.BlockSpec((SEQ_BLK, L * D), lambda i: (i, 0)),
            full((D, 1)),
            full((D, D)), full((1, D)), full((D, D)), full((1, D)),
            full((D, 2 * D)), full((1, 2 * D)),
            full((D, 2 * D)), full((1, 2 * D)),
            full((2 * D, D)), full((1, D)),
        ],
        out_specs=pl.BlockSpec((SEQ_BLK, L, D), lambda i: (i, 0, 0)),
        out_shape=jax.ShapeDtypeStruct((B, L, D), jnp.float32),
    )(seqs3, e13, e2f, b,
      conv1_w, r1(conv1_b), conv2_w, r1(conv2_b),
      up_w, r1(up_b), gate_w, r1(gate_b), down_w, r1(down_b))


def kernel(log_seqs, item_emb, pos_emb, W_item, a_item, W_1, W_2, b,
           co_center, co_neighbor, conv1_w, conv1_b, conv2_w, conv2_b,
           up_w, up_b, gate_w, gate_b, down_w, down_b, adj):
    log_seqs = log_seqs.astype(jnp.int32)
    h, wh1, wh2r = _run_proj(item_emb, W_item, a_item)
    tab1, tab2 = _run_graph(adj, wh1, wh2r, h, co_center, co_neighbor,
                            item_emb)
    idx_flat = log_seqs.reshape(-1)
    g1g, g2g = _run_gather(tab1, tab2, idx_flat)
    seqs, e1, e2 = _run_seq1(g1g, g2g, log_seqs, pos_emb, W_1, W_2)
    return _run_seq2(seqs, e1, e2, b, conv1_w, conv1_b, conv2_w, conv2_b,
                     up_w, up_b, gate_w, gate_b, down_w, down_b)


# speedup vs baseline: 1.4617x; 1.0885x over previous
import functools

import jax
import jax.numpy as jnp
from jax import lax
from jax.experimental import pallas as pl
from jax.experimental.pallas import tpu as pltpu
from jax.experimental.pallas import tpu_sc as plsc

N1 = 5001
D = 64
L = 50
B = 256

ROW_BLK = 256
SEQ_BLK = 8


def _proj_body(emb_ref, embT_ref, Wi_ref, WiT_ref, a1_ref, a2T_ref,
               haug_ref, hmean_ref, wh1_ref, wh2r_ref):
    emb = emb_ref[...]
    h = jnp.dot(emb, Wi_ref[...], preferred_element_type=jnp.float32)
    haug_ref[...] = jnp.concatenate(
        [h, jnp.ones((N1, 1), jnp.float32)], axis=1)
    ones_r = jnp.ones((1, N1), jnp.float32)
    hmean_ref[...] = jnp.dot(ones_r, h,
                             preferred_element_type=jnp.float32) / float(N1)
    wh1_ref[...] = jnp.dot(h, a1_ref[...], preferred_element_type=jnp.float32)
    v = jnp.dot(a2T_ref[...], WiT_ref[...],
                preferred_element_type=jnp.float32)
    wh2r_ref[...] = jnp.dot(v, embT_ref[...],
                            preferred_element_type=jnp.float32)


def _run_proj(item_emb, W_item, a_item):
    emb_T = jnp.transpose(item_emb)
    Wi_T = jnp.transpose(W_item)
    a1 = a_item[:D]
    a2T = jnp.transpose(a_item[D:])
    return pl.pallas_call(
        _proj_body,
        out_shape=(
            jax.ShapeDtypeStruct((N1, D + 1), jnp.float32),
            jax.ShapeDtypeStruct((1, D), jnp.float32),
            jax.ShapeDtypeStruct((N1, 1), jnp.float32),
            jax.ShapeDtypeStruct((1, N1), jnp.float32),
        ),
    )(item_emb, emb_T, W_item, Wi_T, a1, a2T)


def _graph_body(adj_ref, wh1_ref, wh2r_ref, haug_ref, hmean_ref,
                cc_ref, cn_ref, emb_ref, t1_ref, t2_ref):
    a = adj_ref[...]
    e = wh1_ref[...] + wh2r_ref[...]
    e = jnp.maximum(e, 0.01 * e)
    ex = jnp.where(a > 0.0, jnp.exp(e), 0.0)
    haug = haug_ref[...]
    gs = jnp.dot(ex, haug, preferred_element_type=jnp.float32)
    ts = jnp.dot(a, haug, preferred_element_type=jnp.float32)
    s = gs[:, D:]
    rs = ts[:, D:]
    srecip = 1.0 / jnp.where(s > 0.0, s, 1.0)
    gat = jnp.where(s > 0.0, gs[:, :D] * srecip, hmean_ref[...])
    ti = ts[:, :D] * (1.0 / (rs + 1e-8))
    g1 = (jnp.dot(gat, cc_ref[...], preferred_element_type=jnp.float32)
          + jnp.dot(ti, cn_ref[...], preferred_element_type=jnp.float32))
    t1_ref[...] = jnp.concatenate([gat, ti], axis=1)
    t2_ref[...] = jnp.concatenate([g1, emb_ref[...]], axis=1)


def _run_graph(adj, wh1, wh2r, haug, hmean, co_center, co_neighbor, item_emb):
    grid = (pl.cdiv(N1, ROW_BLK),)
    return pl.pallas_call(
        _graph_body,
        grid=grid,
        in_specs=[
            pl.BlockSpec((ROW_BLK, N1), lambda i: (i, 0)),
            pl.BlockSpec((ROW_BLK, 1), lambda i: (i, 0)),
            pl.BlockSpec((1, N1), lambda i: (0, 0)),
            pl.BlockSpec((N1, D + 1), lambda i: (0, 0)),
            pl.BlockSpec((1, D), lambda i: (0, 0)),
            pl.BlockSpec((D, D), lambda i: (0, 0)),
            pl.BlockSpec((D, D), lambda i: (0, 0)),
            pl.BlockSpec((ROW_BLK, D), lambda i: (i, 0)),
        ],
        out_specs=(
            pl.BlockSpec((ROW_BLK, 2 * D), lambda i: (i, 0)),
            pl.BlockSpec((ROW_BLK, 2 * D), lambda i: (i, 0)),
        ),
        out_shape=(
            jax.ShapeDtypeStruct((N1, 2 * D), jnp.float32),
            jax.ShapeDtypeStruct((N1, 2 * D), jnp.float32),
        ),
    )(adj, wh1, wh2r, haug, hmean, co_center, co_neighbor, item_emb)


_NW = 32
_TOK = B * L
_PER_W = _TOK // _NW
_CHUNK = 80
_NCH = _PER_W // _CHUNK


def _gather_body(t0, t1, idx_hbm, o0, o1, idx_v, rows_v, sem):
    nc = 2
    wid = lax.axis_index("s") * nc + lax.axis_index("c")
    pltpu.sync_copy(idx_hbm.at[wid], idx_v)
    for tab, out in ((t0, o0), (t1, o1)):
        handles = [
            pltpu.async_copy(tab.at[idx_v.at[j]],
                             rows_v.at[pl.ds(j * _CHUNK, _CHUNK)], sem)
            for j in range(_NCH)
        ]
        for hd in handles:
            hd.wait()
        pltpu.sync_copy(rows_v, out.at[pl.ds(wid * _PER_W, _PER_W)])


def _run_gather(t1, t2, idx_flat):
    idx3 = idx_flat.reshape(_NW, _NCH, _CHUNK)
    mesh = plsc.VectorSubcoreMesh(core_axis_name="c", subcore_axis_name="s")
    out_t = tuple(jax.ShapeDtypeStruct((_TOK, 2 * D), jnp.float32)
                  for _ in range(2))
    fn = functools.partial(
        pl.kernel,
        mesh=mesh,
        out_type=out_t,
        scratch_types=[
            pltpu.VMEM((_NCH, _CHUNK), jnp.int32),
            pltpu.VMEM((_PER_W, 2 * D), jnp.float32),
            pltpu.SemaphoreType.DMA,
        ],
    )(_gather_body)
    return fn(t1, t2, idx3)


TOK_BLK = 512


def _seq1_body(p1_ref, p2_ref, ls_ref, pos_ref, W1_ref, W2_ref,
               seqs_ref, e1_ref, e2_ref):
    p1 = p1_ref[...]
    p2 = p2_ref[...]
    gat, tr = p1[:, :D], p1[:, D:]
    g1, se = p2[:, :D], p2[:, D:]
    coff = jax.nn.sigmoid(g1)
    seqs = coff * gat + (1.0 - coff) * tr + se
    keep = (ls_ref[...] != 0).astype(jnp.float32)
    sp = seqs + pos_ref[...] * keep
    m1 = jnp.dot(sp, W1_ref[...], preferred_element_type=jnp.float32)
    m2 = jnp.dot(sp, W2_ref[...], preferred_element_type=jnp.float32)
    seqs_ref[...] = seqs
    e1_ref[...] = jnp.exp(-m1)
    e2_ref[...] = jnp.exp(-m2)


def _run_seq1(p1g, p2g, log_seqs, pos_emb, W_1, W_2):
    ls2 = log_seqs.reshape(_TOK, 1)
    pos_t = jnp.tile(pos_emb, (B, 1))
    grid = (_TOK // TOK_BLK,)
    blk = lambda w: pl.BlockSpec((TOK_BLK, w), lambda i: (i, 0))
    full = lambda shape: pl.BlockSpec(shape, lambda i: (0, 0))
    out = jax.ShapeDtypeStruct((_TOK, D), jnp.float32)
    return pl.pallas_call(
        _seq1_body,
        grid=grid,
        in_specs=[blk(2 * D), blk(2 * D), blk(1), blk(D),
                  full((D, D)), full((D, D))],
        out_specs=(blk(D), blk(D), blk(D)),
        out_shape=(out, out, out),
    )(p1g, p2g, ls2, pos_t, W_1, W_2)


def _seq2_body(seqs_ref, e1_ref, e2f_ref, bcol_ref,
               c1w_ref, c1b_ref, c2w_ref, c2b_ref,
               upw_ref, upb_ref, gw_ref, gb_ref, dw_ref, db_ref, out_ref):
    ri = lax.broadcasted_iota(jnp.int32, (SEQ_BLK * L, L), 0)
    ci = lax.broadcasted_iota(jnp.int32, (SEQ_BLK * L, L), 1)
    causal_t = ci <= ri % L
    srow = lax.broadcasted_iota(jnp.int32, (L * D, L), 0)
    jcol = lax.broadcasted_iota(jnp.int32, (L * D, L), 1)
    btile = jnp.tile(bcol_ref[...], (L, 1))
    B3 = jnp.where(srow // D == jcol, 1.0, 0.0) * btile
    tk = lax.broadcasted_iota(jnp.int32, (D, L * D), 0)
    tc = lax.broadcasted_iota(jnp.int32, (D, L * D), 1)
    TILE = jnp.where(tc % D == tk, 1.0, 0.0)
    a2s = []
    for bi in range(SEQ_BLK):
        e1t = jnp.dot(e1_ref[bi], TILE, preferred_element_type=jnp.float32)
        e2t = jnp.broadcast_to(e2f_ref[pl.ds(bi, 1), :], (L, L * D))
        a2s.append(1.0 / (1.0 + e1t * e2t))
    a2a = jnp.concatenate(a2s, axis=0)
    sa = jnp.dot(a2a, B3, preferred_element_type=jnp.float32)
    sa = jnp.where(causal_t, sa, 0.0)
    fins = [
        jnp.dot(sa[bi * L:(bi + 1) * L], seqs_ref[bi],
                preferred_element_type=jnp.float32)
        for bi in range(SEQ_BLK)
    ]
    fin = jnp.concatenate(fins, axis=0)
    hh = jnp.maximum(
        jnp.dot(fin, c1w_ref[...], preferred_element_type=jnp.float32)
        + c1b_ref[...], 0.0)
    hh = jnp.dot(hh, c2w_ref[...], preferred_element_type=jnp.float32) \
        + c2b_ref[...]
    fin = fin + hh
    y_up = jnp.dot(fin, upw_ref[...], preferred_element_type=jnp.float32) \
        + upb_ref[...]
    gate = jnp.maximum(
        jnp.dot(fin, gw_ref[...], preferred_element_type=jnp.float32)
        + gb_ref[...], 0.0)
    dn = jnp.maximum(
        jnp.dot(gate * y_up, dw_ref[...], preferred_element_type=jnp.float32)
        + db_ref[...], 0.0)
    out_ref[...] = (fin + dn).reshape(SEQ_BLK, L, D)


def _run_seq2(seqs, e1, e2, b, conv1_w, conv1_b, conv2_w, conv2_b,
              up_w, up_b, gate_w, gate_b, down_w, down_b):
    seqs3 = seqs.reshape(B, L, D)
    e13 = e1.reshape(B, L, D)
    e2f = e2.reshape(B, L * D)
    r1 = lambda x: x.reshape(1, -1)
    grid = (B // SEQ_BLK,)
    tok = pl.BlockSpec((SEQ_BLK, L, D), lambda i: (i, 0, 0))
    full = lambda shape: pl.BlockSpec(shape, lambda i: tuple(0 for _ in shape))
    return pl.pallas_call(
        _seq2_body,
        grid=grid,
        in_specs=[
            tok, tok,
            pl.BlockSpec((SEQ_BLK, L * D), lambda i: (i, 0)),
            full((D, 1)),
            full((D, D)), full((1, D)), full((D, D)), full((1, D)),
            full((D, 2 * D)), full((1, 2 * D)),
            full((D, 2 * D)), full((1, 2 * D)),
            full((2 * D, D)), full((1, D)),
        ],
        out_specs=pl.BlockSpec((SEQ_BLK, L, D), lambda i: (i, 0, 0)),
        out_shape=jax.ShapeDtypeStruct((B, L, D), jnp.float32),
    )(seqs3, e13, e2f, b,
      conv1_w, r1(conv1_b), conv2_w, r1(conv2_b),
      up_w, r1(up_b), gate_w, r1(gate_b), down_w, r1(down_b))


def kernel(log_seqs, item_emb, pos_emb, W_item, a_item, W_1, W_2, b,
           co_center, co_neighbor, conv1_w, conv1_b, conv2_w, conv2_b,
           up_w, up_b, gate_w, gate_b, down_w, down_b, adj):
    log_seqs = log_seqs.astype(jnp.int32)
    haug, hmean, wh1, wh2r = _run_proj(item_emb, W_item, a_item)
    tab1, tab2 = _run_graph(adj, wh1, wh2r, haug, hmean, co_center,
                            co_neighbor, item_emb)
    idx_flat = log_seqs.reshape(-1)
    g1g, g2g = _run_gather(tab1, tab2, idx_flat)
    seqs, e1, e2 = _run_seq1(g1g, g2g, log_seqs, pos_emb, W_1, W_2)
    return _run_seq2(seqs, e1, e2, b, conv1_w, conv1_b, conv2_w, conv2_b,
                     up_w, up_b, gate_w, gate_b, down_w, down_b)
